# baseline (speedup 1.0000x reference)
import functools

import jax
import jax.numpy as jnp
from jax import lax
from jax.experimental import pallas as pl
from jax.experimental.pallas import tpu as pltpu
from jax.experimental.pallas import tpu_sc as plsc

_N = 50000
_E = 800000
_B = 1024
_H = 128
_NF = 9

_NP = 51200
_CHUNK = 12800
_AROWS = 12928
_ZR = 808
_DR = 800
_GB = 64
_ESS = 3584
_NSS = 14
_NBS = _ESS // _GB
_EPT = _NSS * _ESS
_EP = 16 * _EPT
_PGB = 128

_R = 512
_GRID = _NP // _R

_PRT = _NP // 32
_PNB = _PRT // _PGB
_PTAIL = _PRT - _PNB * _PGB
_PACC = 1152


def _node_mlp(x_p, W1, b1, W2, b2):
    def body(x_ref, w1_ref, b1_ref, w2_ref, b2_ref, o_ref):
        t = jnp.maximum(
            jnp.dot(x_ref[...], w1_ref[...], preferred_element_type=jnp.float32)
            + b1_ref[...], 0.0)
        o_ref[...] = (
            jnp.dot(t, w2_ref[...], preferred_element_type=jnp.float32)
            + b2_ref[...])
    return pl.pallas_call(
        body,
        grid=(_GRID,),
        in_specs=[
            pl.BlockSpec((_R, _NF), lambda i: (i, 0)),
            pl.BlockSpec((_NF, _H), lambda i: (0, 0)),
            pl.BlockSpec((1, _H), lambda i: (0, 0)),
            pl.BlockSpec((_H, _H), lambda i: (0, 0)),
            pl.BlockSpec((1, _H), lambda i: (0, 0)),
        ],
        out_specs=pl.BlockSpec((_R, _H), lambda i: (i, 0)),
        out_shape=jax.ShapeDtypeStruct((_NP, _H), jnp.float32),
    )(x_p, W1, b1.reshape(1, _H), W2, b2.reshape(1, _H))


def _gin_stats(h, agg, W1, b1, W2, b2):
    def body(h_ref, a_ref, w1_ref, b1_ref, w2_ref, b2_ref, y_ref, st_ref,
             acc_ref):
        i = pl.program_id(0)
        u = h_ref[...] + a_ref[...]
        t = jnp.maximum(
            jnp.dot(u, w1_ref[...], preferred_element_type=jnp.float32)
            + b1_ref[...], 0.0)
        y = jnp.maximum(
            jnp.dot(t, w2_ref[...], preferred_element_type=jnp.float32)
            + b2_ref[...], 0.0)
        y_ref[...] = y
        gidx = i * _R + lax.broadcasted_iota(jnp.int32, (_R, _H), 0)
        ym = jnp.where(gidx < _N, y, 0.0)
        ps = ym.reshape(_R // 8, 8, _H).sum(axis=0)
        pq = (ym * ym).reshape(_R // 8, 8, _H).sum(axis=0)

        @pl.when(i == 0)
        def _():
            acc_ref[...] = jnp.zeros_like(acc_ref)

        acc_ref[0:8] += ps
        acc_ref[8:16] += pq
        st_ref[...] = acc_ref[...]

    return pl.pallas_call(
        body,
        grid=(_GRID,),
        in_specs=[
            pl.BlockSpec((_R, _H), lambda i: (i, 0)),
            pl.BlockSpec((_R, _H), lambda i: (i, 0)),
            pl.BlockSpec((_H, _H), lambda i: (0, 0)),
            pl.BlockSpec((1, _H), lambda i: (0, 0)),
            pl.BlockSpec((_H, _H), lambda i: (0, 0)),
            pl.BlockSpec((1, _H), lambda i: (0, 0)),
        ],
        out_specs=[
            pl.BlockSpec((_R, _H), lambda i: (i, 0)),
            pl.BlockSpec((16, _H), lambda i: (0, 0)),
        ],
        out_shape=[
            jax.ShapeDtypeStruct((_NP, _H), jnp.float32),
            jax.ShapeDtypeStruct((16, _H), jnp.float32),
        ],
        scratch_shapes=[pltpu.VMEM((16, _H), jnp.float32)],
    )(h, agg, W1, b1.reshape(1, _H), W2, b2.reshape(1, _H))


def _affine(y, st, g, b):
    def body(y_ref, st_ref, g_ref, b_ref, o_ref):
        sm = st_ref[0:8].sum(axis=0, keepdims=True)
        sq = st_ref[8:16].sum(axis=0, keepdims=True)
        mean = sm / _N
        var = sq / _N - mean * mean
        s = g_ref[...] * lax.rsqrt(var + 1e-5)
        t = b_ref[...] - mean * s
        o_ref[...] = y_ref[...] * s + t

    return pl.pallas_call(
        body,
        grid=(_GRID,),
        in_specs=[
            pl.BlockSpec((_R, _H), lambda i: (i, 0)),
            pl.BlockSpec((16, _H), lambda i: (0, 0)),
            pl.BlockSpec((1, _H), lambda i: (0, 0)),
            pl.BlockSpec((1, _H), lambda i: (0, 0)),
        ],
        out_specs=pl.BlockSpec((_R, _H), lambda i: (i, 0)),
        out_shape=jax.ShapeDtypeStruct((_NP, _H), jnp.float32),
    )(y, st, g.reshape(1, _H), b.reshape(1, _H))


def _head(pool, tf, pe_W1, pe_b1, pe_W2, pe_b2, pe_W3, pe_b3,
          pr_W1, pr_b1, pr_W2, pr_b2, pr_W3, pr_b3):
    def body(pool_ref, tf_ref, pw1, pb1, pw2, pb2, pw3, pb3,
             rw1, rb1, rw2, rb2, rw3, rb3, o_ref):
        mol = pool_ref[0:_B] + pool_ref[_B:2 * _B]
        p = jnp.maximum(
            jnp.dot(tf_ref[...], pw1[...], preferred_element_type=jnp.float32)
            + pb1[...], 0.0)
        p = jnp.maximum(
            jnp.dot(p, pw2[...], preferred_element_type=jnp.float32)
            + pb2[...], 0.0)
        p = jnp.dot(p, pw3[...], preferred_element_type=jnp.float32) + pb3[...]
        comb = jnp.concatenate([mol, p], axis=1)
        o = jnp.maximum(
            jnp.dot(comb, rw1[...], preferred_element_type=jnp.float32)
            + rb1[...], 0.0)
        o = jnp.maximum(
            jnp.dot(o, rw2[...], preferred_element_type=jnp.float32)
            + rb2[...], 0.0)
        o_ref[...] = (
            jnp.dot(o, rw3[...], preferred_element_type=jnp.float32)
            + rb3[...])

    return pl.pallas_call(
        body,
        out_shape=jax.ShapeDtypeStruct((_B, 1), jnp.float32),
    )(pool, tf, pe_W1, pe_b1.reshape(1, _H), pe_W2, pe_b2.reshape(1, _H),
      pe_W3, pe_b3.reshape(1, _H),
      pr_W1, pr_b1.reshape(1, _H), pr_W2, pr_b2.reshape(1, _H // 2),
      pr_W3, pr_b3.reshape(1, 1))


def _build_sc_scatter():
    mesh = plsc.VectorSubcoreMesh(core_axis_name="c", subcore_axis_name="s")

    @functools.partial(
        pl.kernel,
        out_type=jax.ShapeDtypeStruct((_NP, _H), jnp.float32),
        mesh=mesh,
        scratch_types=[
            pltpu.VMEM((_ESS,), jnp.int32),
            pltpu.VMEM((_ESS,), jnp.int32),
            pltpu.VMEM((_GB,), jnp.int32),
            pltpu.VMEM((_GB,), jnp.int32),
            pltpu.VMEM((_GB, _H), jnp.float32),
            pltpu.VMEM((_GB, _H), jnp.float32),
            pltpu.VMEM_SHARED((_AROWS, _H), jnp.float32),
            pltpu.SemaphoreType.DMA,
            pltpu.SemaphoreType.DMA,
        ],
    )
    def ker(h_hbm, srcp_hbm, dstp_hbm, zero_hbm, out_hbm,
            src_st, dst_st, idxw0, idxw1, rows0, rows1, acc, sem0, sem1):
        core = lax.axis_index("c")
        sub = lax.axis_index("s")

        def fill_idx(buf, g):
            for j in range(_GB // 16):
                buf[pl.ds(j * 16, 16)] = dst_st[pl.ds(g * _GB + j * 16, 16)]

        for ci in range(2):
            lo = (core * 2 + ci) * _CHUNK
            for z in range(_ZR // _PGB):
                pltpu.sync_copy(zero_hbm,
                                acc.at[pl.ds(sub * _ZR + z * _PGB, _PGB)])
            pltpu.sync_copy(
                zero_hbm.at[pl.ds(0, _ZR % _PGB)],
                acc.at[pl.ds(sub * _ZR + (_ZR // _PGB) * _PGB, _ZR % _PGB)])
            plsc.subcore_barrier()

            for ss in range(_NSS):
                ebase = sub * _EPT + ss * _ESS
                pltpu.sync_copy(srcp_hbm.at[pl.ds(ebase, _ESS)], src_st)
                pltpu.sync_copy(dstp_hbm.at[pl.ds(ebase, _ESS)], dst_st)

                def xform(v, _):
                    dv = dst_st[pl.ds(v * 16, 16)]
                    sv = src_st[pl.ds(v * 16, 16)]
                    dl = dv - lo
                    m = (dl >= 0) & (dl < _CHUNK)
                    trash = _CHUNK + (dv & 127)
                    dst_st[pl.ds(v * 16, 16)] = jnp.where(m, dl, trash)
                    src_st[pl.ds(v * 16, 16)] = jnp.where(
                        m, sv, jnp.zeros((16,), jnp.int32))
                    return 0

                lax.fori_loop(0, _ESS // 16, xform, 0)

                fill_idx(idxw0, 0)
                pltpu.async_copy(h_hbm.at[src_st.at[pl.ds(0, _GB)]],
                                 rows0, sem0)

                def pair(i, _):
                    g0 = 2 * i
                    g1 = g0 + 1
                    fill_idx(idxw1, g1)
                    pltpu.make_async_copy(h_hbm.at[pl.ds(0, _GB)], rows0,
                                          sem0).wait()
                    pltpu.async_copy(
                        h_hbm.at[src_st.at[pl.ds(g1 * _GB, _GB)]],
                        rows1, sem1)
                    pltpu.sync_copy(rows0, acc.at[idxw0], add=True)

                    @pl.when(g0 + 2 < _NBS)
                    def _():
                        fill_idx(idxw0, g0 + 2)

                    pltpu.make_async_copy(h_hbm.at[pl.ds(0, _GB)], rows1,
                                          sem1).wait()

                    @pl.when(g0 + 2 < _NBS)
                    def _():
                        pltpu.async_copy(
                            h_hbm.at[src_st.at[pl.ds((g0 + 2) * _GB, _GB)]],
                            rows0, sem0)

                    pltpu.sync_copy(rows1, acc.at[idxw1], add=True)
                    return 0

                lax.fori_loop(0, _NBS // 2, pair, 0)

            plsc.subcore_barrier()
            pltpu.sync_copy(acc.at[pl.ds(sub * _DR, _DR)],
                            out_hbm.at[pl.ds(lo + sub * _DR, _DR)])
            plsc.subcore_barrier()

    return ker


def _build_sc_pool():
    mesh = plsc.VectorSubcoreMesh(core_axis_name="c", subcore_axis_name="s")

    @functools.partial(
        pl.kernel,
        out_type=jax.ShapeDtypeStruct((2 * _B, _H), jnp.float32),
        mesh=mesh,
        scratch_types=[
            pltpu.VMEM((_PGB,), jnp.int32),
            pltpu.VMEM((_PGB, _H), jnp.float32),
            pltpu.VMEM_SHARED((_PACC, _H), jnp.float32),
        ],
    )
    def ker(h_hbm, bid_hbm, zero_hbm, out_hbm, idxw, rows, acc):
        core = lax.axis_index("c")
        sub = lax.axis_index("s")
        pltpu.sync_copy(zero_hbm.at[pl.ds(0, _PACC // 16)],
                        acc.at[pl.ds(sub * (_PACC // 16), _PACC // 16)])
        plsc.subcore_barrier()
        rowbase = (core * 16 + sub) * _PRT

        def blk(g, _):
            base = rowbase + g * _PGB
            pltpu.sync_copy(bid_hbm.at[pl.ds(base, _PGB)], idxw)
            pltpu.sync_copy(h_hbm.at[pl.ds(base, _PGB)], rows)
            pltpu.sync_copy(rows, acc.at[idxw], add=True)
            return 0

        lax.fori_loop(0, _PNB, blk, 0)
        base = rowbase + _PNB * _PGB
        pltpu.sync_copy(zero_hbm, rows)
        pltpu.sync_copy(bid_hbm.at[pl.ds(base, _PTAIL)],
                        idxw.at[pl.ds(0, _PTAIL)])
        for j in range(_PTAIL // 16, _PGB // 16):
            idxw[pl.ds(j * 16, 16)] = jnp.full((16,), _B, jnp.int32)
        pltpu.sync_copy(h_hbm.at[pl.ds(base, _PTAIL)],
                        rows.at[pl.ds(0, _PTAIL)])
        pltpu.sync_copy(rows, acc.at[idxw], add=True)
        plsc.subcore_barrier()
        pltpu.sync_copy(acc.at[pl.ds(sub * 64, 64)],
                        out_hbm.at[pl.ds(core * _B + sub * 64, 64)])

    return ker


_SC_SCATTER = _build_sc_scatter()
_SC_POOL = _build_sc_pool()


def kernel(x, edge_index, batch_ids, target_features,
           ne_W1, ne_b1, ne_W2, ne_b2,
           nn1_W1, nn1_b1, nn1_W2, nn1_b2, bn1_g, bn1_b,
           nn2_W1, nn2_b1, nn2_W2, nn2_b2, bn2_g, bn2_b,
           pe_W1, pe_b1, pe_W2, pe_b2, pe_W3, pe_b3,
           pr_W1, pr_b1, pr_W2, pr_b2, pr_W3, pr_b3):
    src_p = jnp.concatenate(
        [edge_index[0], jnp.zeros((_EP - _E,), jnp.int32)])
    dst_p = jnp.concatenate(
        [edge_index[1], jnp.full((_EP - _E,), _NP, jnp.int32)])
    x_p = jnp.pad(x, ((0, _NP - _N), (0, 0)))
    bid_p = jnp.concatenate(
        [batch_ids, jnp.full((_NP - _N,), _B, jnp.int32)])
    zeros = jnp.zeros((_PGB, _H), jnp.float32)

    h0 = _node_mlp(x_p, ne_W1, ne_b1, ne_W2, ne_b2)
    a0 = _SC_SCATTER(h0, src_p, dst_p, zeros)
    y1, st1 = _gin_stats(h0, a0, nn1_W1, nn1_b1, nn1_W2, nn1_b2)
    h1 = _affine(y1, st1, bn1_g, bn1_b)
    a1 = _SC_SCATTER(h1, src_p, dst_p, zeros)
    y2, st2 = _gin_stats(h1, a1, nn2_W1, nn2_b1, nn2_W2, nn2_b2)
    h2 = _affine(y2, st2, bn2_g, bn2_b)
    pool = _SC_POOL(h2, bid_p, zeros)
    return _head(pool, target_features,
                 pe_W1, pe_b1, pe_W2, pe_b2, pe_W3, pe_b3,
                 pr_W1, pr_b1, pr_W2, pr_b2, pr_W3, pr_b3)


# speedup vs baseline: 41.0188x; 41.0188x over previous
import functools

import jax
import jax.numpy as jnp
from jax import lax
from jax.experimental import pallas as pl
from jax.experimental.pallas import tpu as pltpu
from jax.experimental.pallas import tpu_sc as plsc

_N = 50000
_E = 800000
_B = 1024
_H = 128
_NF = 9

_NP = 51200
_CHUNK = 12800
_AROWS = 12928
_ZR = 808
_DR = 800
_GB = 64
_ESS = 3584
_NSS = 14
_NBS = _ESS // _GB
_EPT = _NSS * _ESS
_EP = 16 * _EPT
_PGB = 128

_R = 512
_GRID = _NP // _R

_PRT = _NP // 32
_PNB = _PRT // _PGB
_PTAIL = _PRT - _PNB * _PGB
_PACC = 1152


def _node_mlp(x_p, W1, b1, W2, b2):
    def body(x_ref, w1_ref, b1_ref, w2_ref, b2_ref, o_ref):
        t = jnp.maximum(
            jnp.dot(x_ref[...], w1_ref[...], preferred_element_type=jnp.float32)
            + b1_ref[...], 0.0)
        o_ref[...] = (
            jnp.dot(t, w2_ref[...], preferred_element_type=jnp.float32)
            + b2_ref[...])
    return pl.pallas_call(
        body,
        grid=(_GRID,),
        in_specs=[
            pl.BlockSpec((_R, _NF), lambda i: (i, 0)),
            pl.BlockSpec((_NF, _H), lambda i: (0, 0)),
            pl.BlockSpec((1, _H), lambda i: (0, 0)),
            pl.BlockSpec((_H, _H), lambda i: (0, 0)),
            pl.BlockSpec((1, _H), lambda i: (0, 0)),
        ],
        out_specs=pl.BlockSpec((_R, _H), lambda i: (i, 0)),
        out_shape=jax.ShapeDtypeStruct((_NP, _H), jnp.float32),
    )(x_p, W1, b1.reshape(1, _H), W2, b2.reshape(1, _H))


def _gin_stats(h, agg, W1, b1, W2, b2):
    def body(h_ref, a_ref, w1_ref, b1_ref, w2_ref, b2_ref, y_ref, st_ref,
             acc_ref):
        i = pl.program_id(0)
        u = h_ref[...] + a_ref[...]
        t = jnp.maximum(
            jnp.dot(u, w1_ref[...], preferred_element_type=jnp.float32)
            + b1_ref[...], 0.0)
        y = jnp.maximum(
            jnp.dot(t, w2_ref[...], preferred_element_type=jnp.float32)
            + b2_ref[...], 0.0)
        y_ref[...] = y
        gidx = i * _R + lax.broadcasted_iota(jnp.int32, (_R, _H), 0)
        ym = jnp.where(gidx < _N, y, 0.0)
        ps = ym.reshape(_R // 8, 8, _H).sum(axis=0)
        pq = (ym * ym).reshape(_R // 8, 8, _H).sum(axis=0)

        @pl.when(i == 0)
        def _():
            acc_ref[...] = jnp.zeros_like(acc_ref)

        acc_ref[0:8] += ps
        acc_ref[8:16] += pq
        st_ref[...] = acc_ref[...]

    return pl.pallas_call(
        body,
        grid=(_GRID,),
        in_specs=[
            pl.BlockSpec((_R, _H), lambda i: (i, 0)),
            pl.BlockSpec((_R, _H), lambda i: (i, 0)),
            pl.BlockSpec((_H, _H), lambda i: (0, 0)),
            pl.BlockSpec((1, _H), lambda i: (0, 0)),
            pl.BlockSpec((_H, _H), lambda i: (0, 0)),
            pl.BlockSpec((1, _H), lambda i: (0, 0)),
        ],
        out_specs=[
            pl.BlockSpec((_R, _H), lambda i: (i, 0)),
            pl.BlockSpec((16, _H), lambda i: (0, 0)),
        ],
        out_shape=[
            jax.ShapeDtypeStruct((_NP, _H), jnp.float32),
            jax.ShapeDtypeStruct((16, _H), jnp.float32),
        ],
        scratch_shapes=[pltpu.VMEM((16, _H), jnp.float32)],
    )(h, agg, W1, b1.reshape(1, _H), W2, b2.reshape(1, _H))


def _affine(y, st, g, b):
    def body(y_ref, st_ref, g_ref, b_ref, o_ref):
        sm = st_ref[0:8].sum(axis=0, keepdims=True)
        sq = st_ref[8:16].sum(axis=0, keepdims=True)
        mean = sm / _N
        var = sq / _N - mean * mean
        s = g_ref[...] * lax.rsqrt(var + 1e-5)
        t = b_ref[...] - mean * s
        o_ref[...] = y_ref[...] * s + t

    return pl.pallas_call(
        body,
        grid=(_GRID,),
        in_specs=[
            pl.BlockSpec((_R, _H), lambda i: (i, 0)),
            pl.BlockSpec((16, _H), lambda i: (0, 0)),
            pl.BlockSpec((1, _H), lambda i: (0, 0)),
            pl.BlockSpec((1, _H), lambda i: (0, 0)),
        ],
        out_specs=pl.BlockSpec((_R, _H), lambda i: (i, 0)),
        out_shape=jax.ShapeDtypeStruct((_NP, _H), jnp.float32),
    )(y, st, g.reshape(1, _H), b.reshape(1, _H))


def _head(pool, tf, pe_W1, pe_b1, pe_W2, pe_b2, pe_W3, pe_b3,
          pr_W1, pr_b1, pr_W2, pr_b2, pr_W3, pr_b3):
    def body(pool_ref, tf_ref, pw1, pb1, pw2, pb2, pw3, pb3,
             rw1, rb1, rw2, rb2, rw3, rb3, o_ref):
        mol = pool_ref[0:_B] + pool_ref[_B:2 * _B]
        p = jnp.maximum(
            jnp.dot(tf_ref[...], pw1[...], preferred_element_type=jnp.float32)
            + pb1[...], 0.0)
        p = jnp.maximum(
            jnp.dot(p, pw2[...], preferred_element_type=jnp.float32)
            + pb2[...], 0.0)
        p = jnp.dot(p, pw3[...], preferred_element_type=jnp.float32) + pb3[...]
        comb = jnp.concatenate([mol, p], axis=1)
        o = jnp.maximum(
            jnp.dot(comb, rw1[...], preferred_element_type=jnp.float32)
            + rb1[...], 0.0)
        o = jnp.maximum(
            jnp.dot(o, rw2[...], preferred_element_type=jnp.float32)
            + rb2[...], 0.0)
        o_ref[...] = (
            jnp.dot(o, rw3[...], preferred_element_type=jnp.float32)
            + rb3[...])

    return pl.pallas_call(
        body,
        out_shape=jax.ShapeDtypeStruct((_B, 1), jnp.float32),
    )(pool, tf, pe_W1, pe_b1.reshape(1, _H), pe_W2, pe_b2.reshape(1, _H),
      pe_W3, pe_b3.reshape(1, _H),
      pr_W1, pr_b1.reshape(1, _H), pr_W2, pr_b2.reshape(1, _H // 2),
      pr_W3, pr_b3.reshape(1, 1))


def _build_sc_scatter():
    mesh = plsc.VectorSubcoreMesh(core_axis_name="c", subcore_axis_name="s")

    @functools.partial(
        pl.kernel,
        out_type=jax.ShapeDtypeStruct((_NP, _H), jnp.float32),
        mesh=mesh,
        scratch_types=[
            pltpu.VMEM((_ESS,), jnp.int32),
            pltpu.VMEM((_ESS,), jnp.int32),
            pltpu.VMEM((_GB,), jnp.int32),
            pltpu.VMEM((_GB,), jnp.int32),
            pltpu.VMEM((_GB, _H), jnp.float32),
            pltpu.VMEM((_GB, _H), jnp.float32),
            pltpu.VMEM_SHARED((_AROWS, _H), jnp.float32),
            pltpu.SemaphoreType.DMA,
            pltpu.SemaphoreType.DMA,
        ],
    )
    def ker(h_hbm, srcp_hbm, dstp_hbm, zero_hbm, out_hbm,
            src_st, dst_st, idxw0, idxw1, rows0, rows1, acc, sem0, sem1):
        core = lax.axis_index("c")
        sub = lax.axis_index("s")

        def fill_idx(buf, g):
            for j in range(_GB // 16):
                buf[pl.ds(j * 16, 16)] = dst_st[pl.ds(g * _GB + j * 16, 16)]

        for ci in range(2):
            lo = (core * 2 + ci) * _CHUNK
            for z in range(_ZR // _PGB):
                pltpu.sync_copy(zero_hbm,
                                acc.at[pl.ds(sub * _ZR + z * _PGB, _PGB)])
            pltpu.sync_copy(
                zero_hbm.at[pl.ds(0, _ZR % _PGB)],
                acc.at[pl.ds(sub * _ZR + (_ZR // _PGB) * _PGB, _ZR % _PGB)])
            plsc.subcore_barrier()

            for ss in range(_NSS):
                ebase = sub * _EPT + ss * _ESS
                pltpu.sync_copy(srcp_hbm.at[pl.ds(ebase, _ESS)], src_st)
                pltpu.sync_copy(dstp_hbm.at[pl.ds(ebase, _ESS)], dst_st)

                def xform(v, _):
                    dv = dst_st[pl.ds(v * 16, 16)]
                    dl = dv - lo
                    m = (dl >= 0) & (dl < _CHUNK)
                    dst_st[pl.ds(v * 16, 16)] = jnp.where(m, dl, _CHUNK)
                    return 0

                lax.fori_loop(0, _ESS // 16, xform, 0)

                fill_idx(idxw0, 0)
                pltpu.async_copy(h_hbm.at[src_st.at[pl.ds(0, _GB)]],
                                 rows0, sem0)

                def pair(i, _):
                    g0 = 2 * i
                    g1 = g0 + 1
                    fill_idx(idxw1, g1)
                    pltpu.make_async_copy(h_hbm.at[pl.ds(0, _GB)], rows0,
                                          sem0).wait()
                    pltpu.async_copy(
                        h_hbm.at[src_st.at[pl.ds(g1 * _GB, _GB)]],
                        rows1, sem1)
                    pltpu.sync_copy(rows0, acc.at[idxw0], add=True)

                    @pl.when(g0 + 2 < _NBS)
                    def _():
                        fill_idx(idxw0, g0 + 2)

                    pltpu.make_async_copy(h_hbm.at[pl.ds(0, _GB)], rows1,
                                          sem1).wait()

                    @pl.when(g0 + 2 < _NBS)
                    def _():
                        pltpu.async_copy(
                            h_hbm.at[src_st.at[pl.ds((g0 + 2) * _GB, _GB)]],
                            rows0, sem0)

                    pltpu.sync_copy(rows1, acc.at[idxw1], add=True)
                    return 0

                lax.fori_loop(0, _NBS // 2, pair, 0)

            plsc.subcore_barrier()
            pltpu.sync_copy(acc.at[pl.ds(sub * _DR, _DR)],
                            out_hbm.at[pl.ds(lo + sub * _DR, _DR)])
            plsc.subcore_barrier()

    return ker


def _build_sc_pool():
    mesh = plsc.VectorSubcoreMesh(core_axis_name="c", subcore_axis_name="s")

    @functools.partial(
        pl.kernel,
        out_type=jax.ShapeDtypeStruct((2 * _B, _H), jnp.float32),
        mesh=mesh,
        scratch_types=[
            pltpu.VMEM((_PGB,), jnp.int32),
            pltpu.VMEM((_PGB, _H), jnp.float32),
            pltpu.VMEM_SHARED((_PACC, _H), jnp.float32),
        ],
    )
    def ker(h_hbm, bid_hbm, zero_hbm, out_hbm, idxw, rows, acc):
        core = lax.axis_index("c")
        sub = lax.axis_index("s")
        pltpu.sync_copy(zero_hbm.at[pl.ds(0, _PACC // 16)],
                        acc.at[pl.ds(sub * (_PACC // 16), _PACC // 16)])
        plsc.subcore_barrier()
        rowbase = (core * 16 + sub) * _PRT

        def blk(g, _):
            base = rowbase + g * _PGB
            pltpu.sync_copy(bid_hbm.at[pl.ds(base, _PGB)], idxw)
            pltpu.sync_copy(h_hbm.at[pl.ds(base, _PGB)], rows)
            pltpu.sync_copy(rows, acc.at[idxw], add=True)
            return 0

        lax.fori_loop(0, _PNB, blk, 0)
        base = rowbase + _PNB * _PGB
        pltpu.sync_copy(zero_hbm, rows)
        pltpu.sync_copy(bid_hbm.at[pl.ds(base, _PTAIL)],
                        idxw.at[pl.ds(0, _PTAIL)])
        for j in range(_PTAIL // 16, _PGB // 16):
            idxw[pl.ds(j * 16, 16)] = jnp.full((16,), _B, jnp.int32)
        pltpu.sync_copy(h_hbm.at[pl.ds(base, _PTAIL)],
                        rows.at[pl.ds(0, _PTAIL)])
        pltpu.sync_copy(rows, acc.at[idxw], add=True)
        plsc.subcore_barrier()
        pltpu.sync_copy(acc.at[pl.ds(sub * 64, 64)],
                        out_hbm.at[pl.ds(core * _B + sub * 64, 64)])

    return ker


_SC_SCATTER = _build_sc_scatter()
_SC_POOL = _build_sc_pool()


def kernel(x, edge_index, batch_ids, target_features,
           ne_W1, ne_b1, ne_W2, ne_b2,
           nn1_W1, nn1_b1, nn1_W2, nn1_b2, bn1_g, bn1_b,
           nn2_W1, nn2_b1, nn2_W2, nn2_b2, bn2_g, bn2_b,
           pe_W1, pe_b1, pe_W2, pe_b2, pe_W3, pe_b3,
           pr_W1, pr_b1, pr_W2, pr_b2, pr_W3, pr_b3):
    src_p = jnp.concatenate(
        [edge_index[0], jnp.zeros((_EP - _E,), jnp.int32)])
    dst_p = jnp.concatenate(
        [edge_index[1], jnp.full((_EP - _E,), _NP, jnp.int32)])
    x_p = jnp.pad(x, ((0, _NP - _N), (0, 0)))
    bid_p = jnp.concatenate(
        [batch_ids, jnp.full((_NP - _N,), _B, jnp.int32)])
    zeros = jnp.zeros((_PGB, _H), jnp.float32)

    h0 = _node_mlp(x_p, ne_W1, ne_b1, ne_W2, ne_b2)
    a0 = _SC_SCATTER(h0, src_p, dst_p, zeros)
    y1, st1 = _gin_stats(h0, a0, nn1_W1, nn1_b1, nn1_W2, nn1_b2)
    h1 = _affine(y1, st1, bn1_g, bn1_b)
    a1 = _SC_SCATTER(h1, src_p, dst_p, zeros)
    y2, st2 = _gin_stats(h1, a1, nn2_W1, nn2_b1, nn2_W2, nn2_b2)
    h2 = _affine(y2, st2, bn2_g, bn2_b)
    pool = _SC_POOL(h2, bid_p, zeros)
    return _head(pool, target_features,
                 pe_W1, pe_b1, pe_W2, pe_b2, pe_W3, pe_b3,
                 pr_W1, pr_b1, pr_W2, pr_b2, pr_W3, pr_b3)
